# baseline (speedup 1.0000x reference)
import functools

import jax
import jax.numpy as jnp
from jax import lax
from jax.experimental import pallas as pl
from jax.experimental.pallas import tpu as pltpu
from jax.experimental.pallas import tpu_sc as plsc

_N_NODES = 2048
_N_EDGES = 32768
_D_E = 16
_N_CORES = 2
_N_WORKERS = 16 * _N_CORES
_WIN = _N_NODES // _N_WORKERS
_CH = 128

_mesh = plsc.VectorSubcoreMesh(
    core_axis_name="c", subcore_axis_name="s", num_cores=_N_CORES)


def _offsets_body(rt_ref, out_ref):
    bnd = lax.broadcasted_iota(jnp.int32, (128, 128), 1) * _WIN
    acc = jnp.zeros((128, 128), jnp.int32)
    for i in range(256):
        blk = lax.broadcast_in_dim(
            rt_ref[:, pl.ds(i, 1)], (128, 128), (0, 1))
        acc = acc + jnp.where(blk < bnd, 1, 0)
    out_ref[...] = jnp.sum(acc, axis=0, keepdims=True)


@functools.partial(
    pl.kernel,
    out_type=jax.ShapeDtypeStruct((_N_NODES, _D_E), jnp.float32),
    mesh=_mesh,
    compiler_params=pltpu.CompilerParams(needs_layout_passes=False),
    scratch_types=[
        pltpu.VMEM((128,), jnp.int32),
        pltpu.VMEM((_CH, _D_E), jnp.float32),
        pltpu.VMEM((_CH, _D_E), jnp.float32),
        pltpu.VMEM((_CH,), jnp.int32),
        pltpu.VMEM((_CH,), jnp.int32),
        pltpu.VMEM((_WIN, _D_E), jnp.float32),
        pltpu.SemaphoreType.DMA,
        pltpu.SemaphoreType.DMA,
        pltpu.SemaphoreType.DMA,
        pltpu.SemaphoreType.DMA,
    ],
)
def _seg_sum(e_hbm, recv_hbm, offs_hbm, out_hbm, offs_v, e_v0, e_v1,
             rc_v0, rc_v1, acc, sem_e0, sem_e1, sem_r0, sem_r1):
    c = lax.axis_index("c")
    s = lax.axis_index("s")
    wid = c * 16 + s
    lo = wid * _WIN
    hi = lo + _WIN

    pltpu.sync_copy(offs_hbm, offs_v)
    start = plsc.load_gather(offs_v, [jnp.full((16,), wid, jnp.int32)])[0]
    end = plsc.load_gather(offs_v, [jnp.full((16,), wid + 1, jnp.int32)])[0]

    zrow = jnp.zeros((16,), jnp.float32)

    def zero_body(j, _):
        acc[j] = zrow
        return 0

    lax.fori_loop(0, _WIN, zero_body, 0)

    col = lax.iota(jnp.int32, 16)
    c0 = start // _CH
    c1 = (end + _CH - 1) // _CH

    def _start(k, ebuf, rbuf, sem_e, sem_r):
        pltpu.async_copy(e_hbm.at[pl.ds(k * _CH, _CH)], ebuf, sem_e)
        pltpu.async_copy(recv_hbm.at[pl.ds(k * _CH, _CH)], rbuf, sem_r)

    def _process(k, ebuf, rbuf, sem_e, sem_r):
        pltpu.make_async_copy(
            recv_hbm.at[pl.ds(k * _CH, _CH)], rbuf, sem_r).wait()
        pltpu.make_async_copy(
            e_hbm.at[pl.ds(k * _CH, _CH)], ebuf, sem_e).wait()
        for j0 in range(0, _CH, 4):
            rs = [plsc.load_gather(rbuf, [jnp.full((16,), j0 + t, jnp.int32)])
                  for t in range(4)]
            rows = [ebuf[j0 + t] for t in range(4)]
            for t in range(4):
                mask = (rs[t] >= lo) & (rs[t] < hi)
                plsc.addupdate_scatter(acc, [rs[t] - lo, col], rows[t],
                                       mask=mask)

    @pl.when(c0 < c1)
    def _():
        _start(c0, e_v0, rc_v0, sem_e0, sem_r0)

    def chunk_body(k, _):
        even = ((k - c0) % 2) == 0

        @pl.when(k + 1 < c1)
        def _():
            @pl.when(even)
            def _():
                _start(k + 1, e_v1, rc_v1, sem_e1, sem_r1)

            @pl.when(jnp.logical_not(even))
            def _():
                _start(k + 1, e_v0, rc_v0, sem_e0, sem_r0)

        @pl.when(even)
        def _():
            _process(k, e_v0, rc_v0, sem_e0, sem_r0)

        @pl.when(jnp.logical_not(even))
        def _():
            _process(k, e_v1, rc_v1, sem_e1, sem_r1)

        return 0

    lax.fori_loop(c0, c1, chunk_body, 0)

    pltpu.sync_copy(acc, out_hbm.at[pl.ds(lo, _WIN)])


def _combine_body(h_ref, wl_ref, esum_ref, wr_ref, u_ref, out_ref):
    gl = lax.dot_general(
        h_ref[...], wl_ref[...], (((1,), (1,)), ((), ())),
        preferred_element_type=jnp.float32,
    )
    agg = lax.dot_general(
        esum_ref[...], wr_ref[...], (((1,), (1,)), ((), ())),
        preferred_element_type=jnp.float32,
    )
    out_ref[...] = jnp.concatenate([gl, agg, u_ref[...]], axis=1)


def kernel(h, e, receivers, u, W_l, W_r, W_attn):
    del W_attn
    n_nodes = h.shape[0]
    recv = receivers.astype(jnp.int32)
    recv_t = recv.reshape(_N_EDGES // 128, 128).T
    offs = pl.pallas_call(
        _offsets_body,
        out_shape=jax.ShapeDtypeStruct((1, 128), jnp.int32),
    )(recv_t)
    esum = _seg_sum(e, recv, offs.reshape(128))
    out = pl.pallas_call(
        _combine_body,
        out_shape=jax.ShapeDtypeStruct(
            (n_nodes, W_l.shape[0] + W_r.shape[0] + u.shape[1]), jnp.float32),
    )(h, W_l, esum, W_r, u)
    return out
